# Initial kernel scaffold; baseline (speedup 1.0000x reference)
import jax
import jax.numpy as jnp
from jax.experimental import pallas as pl


def kernel(x, norm_g, router_w, W_gu, b_gu, W_d, b_d):
    raise NotImplementedError("write your pallas kernel here")


# speedup vs baseline: 16.7284x; 16.7284x over previous
import functools
import jax
import jax.numpy as jnp
from jax.experimental import pallas as pl
from jax.experimental.pallas import tpu as pltpu

NUM_EXPERTS = 8
TOP_K = 2
HIDDEN = 1024
INTER = 2048
EPS = 1e-6
TI = 512
IB = INTER // TI


def _moe_body(x_ref, g_ref, rw_ref, wg_ref, wl_ref, bgu_ref, wd_ref, bd_ref,
              out_ref, tf_ref, aff_ref):
    e = pl.program_id(0)
    i = pl.program_id(1)
    first = jnp.logical_and(e == 0, i == 0)

    @pl.when(first)
    def _prologue():
        xv = x_ref[...]
        ms = jnp.mean(xv * xv, axis=-1, keepdims=True)
        t = xv * jax.lax.rsqrt(ms + EPS) * g_ref[...]
        tf_ref[...] = t.astype(jnp.bfloat16)
        logits = jnp.dot(t.astype(jnp.bfloat16),
                         rw_ref[...].astype(jnp.bfloat16),
                         preferred_element_type=jnp.float32)
        m = jnp.max(logits, axis=-1, keepdims=True)
        p = jnp.exp(logits - m)
        probs = p / jnp.sum(p, axis=-1, keepdims=True)
        cols = jax.lax.broadcasted_iota(jnp.int32, probs.shape, 1)
        m1 = jnp.max(probs, axis=-1, keepdims=True)
        idx1 = jnp.min(jnp.where(probs == m1, cols, NUM_EXPERTS),
                       axis=-1, keepdims=True)
        probs2 = jnp.where(cols == idx1, -jnp.inf, probs)
        m2 = jnp.max(probs2, axis=-1, keepdims=True)
        idx2 = jnp.min(jnp.where(probs2 == m2, cols, NUM_EXPERTS),
                       axis=-1, keepdims=True)
        denom = m1 + m2
        aff_ref[...] = (jnp.where(cols == idx1, m1, 0.0)
                        + jnp.where(cols == idx2, m2, 0.0)) / denom

    xb = tf_ref[...]
    bgu = bgu_ref[0]
    hg = (jnp.dot(xb, wg_ref[0], preferred_element_type=jnp.float32)
          + bgu[0:1, :])
    hl = (jnp.dot(xb, wl_ref[0], preferred_element_type=jnp.float32)
          + bgu[1:2, :])
    hg = jnp.minimum(hg, 7.0)
    hl = jnp.clip(hl, -7.0, 7.0)
    act = hg * jax.nn.sigmoid(1.702 * hg) * (hl + 1.0)
    part = jnp.dot(act.astype(jnp.bfloat16), wd_ref[0],
                   preferred_element_type=jnp.float32)
    lane = jax.lax.broadcasted_iota(jnp.int32, aff_ref.shape, 1)
    gate = jnp.sum(jnp.where(lane == e, aff_ref[...], 0.0),
                   axis=-1, keepdims=True)
    contrib = gate * part
    contrib = contrib + jnp.where(i == 0, 1.0, 0.0) * (gate * bd_ref[0])
    prev = jnp.where(first, 0.0, out_ref[...])
    out_ref[...] = prev + contrib


@functools.partial(jax.jit, static_argnames=())
def kernel(x, norm_g, router_w, W_gu, b_gu, W_d, b_d):
    b, s, h = x.shape
    T = b * s
    xf = x.reshape(T, h)
    gm = norm_g.reshape(1, h)
    wgu4 = W_gu.reshape(NUM_EXPERTS, h, INTER, 2)
    W_g = wgu4[..., 0].astype(jnp.bfloat16)
    W_l = wgu4[..., 1].astype(jnp.bfloat16)
    bgu2 = jnp.swapaxes(b_gu.reshape(NUM_EXPERTS, INTER, 2), 1, 2)
    W_d_bf = W_d.astype(jnp.bfloat16)

    out = pl.pallas_call(
        _moe_body,
        grid=(NUM_EXPERTS, IB),
        in_specs=[
            pl.BlockSpec((T, h), lambda e, i: (0, 0)),
            pl.BlockSpec((1, h), lambda e, i: (0, 0)),
            pl.BlockSpec((h, NUM_EXPERTS), lambda e, i: (0, 0)),
            pl.BlockSpec((1, h, TI), lambda e, i: (e, 0, i)),
            pl.BlockSpec((1, h, TI), lambda e, i: (e, 0, i)),
            pl.BlockSpec((1, 2, TI), lambda e, i: (e, 0, i)),
            pl.BlockSpec((1, TI, h), lambda e, i: (e, i, 0)),
            pl.BlockSpec((1, 1, h), lambda e, i: (e, 0, 0)),
        ],
        out_specs=pl.BlockSpec((T, h), lambda e, i: (0, 0)),
        out_shape=jax.ShapeDtypeStruct((T, h), jnp.float32),
        scratch_shapes=[
            pltpu.VMEM((T, h), jnp.bfloat16),
            pltpu.VMEM((T, NUM_EXPERTS), jnp.float32),
        ],
        compiler_params=pltpu.CompilerParams(
            dimension_semantics=("arbitrary", "arbitrary"),
        ),
    )(xf, gm, router_w, W_g, W_l, bgu2, W_d_bf,
      b_d.reshape(NUM_EXPERTS, 1, h))
    return out.reshape(b, s, h)
